# baseline (speedup 1.0000x reference)
import functools

import jax
import jax.numpy as jnp
from jax import lax
from jax.experimental import pallas as pl
from jax.experimental.pallas import tpu as pltpu
from jax.experimental.pallas import tpu_sc as plsc

DIM = 64
NUM_CORES = 2
NUM_SUBCORES = 16
NUM_WORKERS = NUM_CORES * NUM_SUBCORES
LANES = 16
IDX_MAX = 128
SCALE = 8.0
EPS = 1e-5


def _rsqrt(v):
    i = lax.bitcast_convert_type(v, jnp.int32)
    i = 0x5F3759DF - lax.shift_right_logical(i, 1)
    y = lax.bitcast_convert_type(i, jnp.float32)
    half = 0.5 * v
    for _ in range(3):
        y = y * (1.5 - half * y * y)
    return y


def _make_kernel(batch, seqlen):
    rows_per_worker = batch // NUM_WORKERS
    outer_iters = rows_per_worker // 2
    segs = []
    off = 0
    while off < seqlen:
        n = min(IDX_MAX, seqlen - off)
        segs.append((off, n))
        off += n
    mesh = plsc.VectorSubcoreMesh(core_axis_name="c", subcore_axis_name="s")

    @functools.partial(
        pl.kernel,
        mesh=mesh,
        compiler_params=pltpu.CompilerParams(
            needs_layout_passes=False, use_tc_tiling_on_sc=False
        ),
        out_type=jax.ShapeDtypeStruct((batch, seqlen, DIM), jnp.float32),
        scratch_types=[
            pltpu.VMEM((rows_per_worker, seqlen), jnp.int32),
            pltpu.VMEM((2, seqlen, DIM), jnp.float32),
            pltpu.VMEM((2, seqlen, DIM), jnp.float32),
            pltpu.VMEM((seqlen, DIM), jnp.float32),
            pltpu.VMEM((DIM,), jnp.float32),
            pltpu.VMEM((DIM,), jnp.float32),
            pltpu.SemaphoreType.DMA,
            pltpu.SemaphoreType.DMA,
            pltpu.SemaphoreType.DMA,
            pltpu.SemaphoreType.DMA,
        ],
    )
    def emb_kernel(seq_hbm, tok_hbm, pos_hbm, gamma_hbm, beta_hbm, out_hbm,
                   idx_all, rows, outb, pos_v, g_v, b_v,
                   sem_g0, sem_g1, sem_o0, sem_o1):
        wid = lax.axis_index("s") * NUM_CORES + lax.axis_index("c")
        row_base = wid * rows_per_worker

        pltpu.sync_copy(seq_hbm.at[pl.ds(row_base, rows_per_worker)], idx_all)
        pltpu.sync_copy(pos_hbm, pos_v)
        pltpu.sync_copy(gamma_hbm, g_v)
        pltpu.sync_copy(beta_hbm, b_v)

        g4 = [g_v[pl.ds(j * LANES, LANES)] for j in range(4)]
        b4 = [b_v[pl.ds(j * LANES, LANES)] for j in range(4)]

        sems_g = (sem_g0, sem_g1)
        sems_o = (sem_o0, sem_o1)

        def gather_copies(r, buf, sem):
            return [
                pltpu.make_async_copy(
                    tok_hbm.at[idx_all.at[r, pl.ds(o, n)]],
                    rows.at[buf, pl.ds(o, n)],
                    sem,
                )
                for o, n in segs
            ]

        def fire_gather(r, buf, sem):
            for cp in gather_copies(r, buf, sem):
                cp.start()

        def wait_gather(r, buf, sem):
            for cp in gather_copies(r, buf, sem):
                cp.wait()

        def store_copy(bi, buf, sem):
            return pltpu.make_async_copy(outb.at[buf], out_hbm.at[bi], sem)

        fire_gather(0, 0, sem_g0)

        def outer(g, _):
            for b in range(2):
                r = g * 2 + b
                nb = 1 - b
                if b == 0:
                    fire_gather(r + 1, nb, sems_g[nb])
                else:
                    @pl.when(g < outer_iters - 1)
                    def _():
                        fire_gather(r + 1, nb, sems_g[nb])
                wait_gather(r, b, sems_g[b])

                bi = row_base + r

                @pl.when(g > 0)
                def _():
                    store_copy(bi, b, sems_o[b]).wait()

                @plsc.parallel_loop(0, seqlen, 1, unroll=8)
                def tok_body(t):
                    x = [rows[b, t, pl.ds(j * LANES, LANES)] * SCALE
                         + pos_v[t, pl.ds(j * LANES, LANES)]
                         for j in range(4)]
                    s = (x[0] + x[1]) + (x[2] + x[3])
                    sq = x[0] * x[0] + (x[1] * x[1] + (x[2] * x[2] + x[3] * x[3]))
                    mean = jnp.sum(s) * (1.0 / DIM)
                    var = jnp.sum(sq) * (1.0 / DIM) - mean * mean
                    inv = _rsqrt(var + EPS)
                    cc = mean * inv
                    for j in range(4):
                        a = inv * g4[j]
                        outb[b, t, pl.ds(j * LANES, LANES)] = (
                            x[j] * a + (b4[j] - cc * g4[j])
                        )

                store_copy(bi, b, sems_o[b]).start()
            return 0

        lax.fori_loop(0, outer_iters, outer, 0)

        for b in range(2):
            store_copy(row_base + b, b, sems_o[b]).wait()

    return emb_kernel


@jax.jit
def kernel(seq, tok_table, pos_table, gamma, beta):
    b, s = seq.shape
    return _make_kernel(b, s)(
        seq.astype(jnp.int32), tok_table, pos_table, gamma, beta
    )


# speedup vs baseline: 1.3709x; 1.3709x over previous
import functools

import jax
import jax.numpy as jnp
from jax import lax
from jax.experimental import pallas as pl
from jax.experimental.pallas import tpu as pltpu
from jax.experimental.pallas import tpu_sc as plsc

DIM = 64
NUM_CORES = 2
NUM_SUBCORES = 16
NUM_WORKERS = NUM_CORES * NUM_SUBCORES
LANES = 16
IDX_MAX = 128
SCALE = 8.0
EPS = 1e-5


def _rsqrt(v):
    i = lax.bitcast_convert_type(v, jnp.int32)
    i = 0x5F3759DF - lax.shift_right_logical(i, 1)
    y = lax.bitcast_convert_type(i, jnp.float32)
    half = 0.5 * v
    for _ in range(3):
        y = y * (1.5 - half * y * y)
    return y


def _make_kernel(batch, seqlen):
    rows_per_worker = batch // NUM_WORKERS
    outer_iters = rows_per_worker // 2
    segs = []
    off = 0
    while off < seqlen:
        n = min(IDX_MAX, seqlen - off)
        segs.append((off, n))
        off += n
    mesh = plsc.VectorSubcoreMesh(core_axis_name="c", subcore_axis_name="s")

    @functools.partial(
        pl.kernel,
        mesh=mesh,
        compiler_params=pltpu.CompilerParams(
            needs_layout_passes=False, use_tc_tiling_on_sc=False
        ),
        out_type=jax.ShapeDtypeStruct((batch, seqlen, DIM), jnp.float32),
        scratch_types=[
            pltpu.VMEM((rows_per_worker, seqlen), jnp.int32),
            pltpu.VMEM((2, seqlen, DIM), jnp.float32),
            pltpu.VMEM((2, seqlen, DIM), jnp.float32),
            pltpu.VMEM((seqlen, DIM), jnp.float32),
            pltpu.VMEM((DIM,), jnp.float32),
            pltpu.VMEM((DIM,), jnp.float32),
            pltpu.SemaphoreType.DMA,
            pltpu.SemaphoreType.DMA,
            pltpu.SemaphoreType.DMA,
            pltpu.SemaphoreType.DMA,
        ],
    )
    def emb_kernel(seq_hbm, tok_hbm, pos_hbm, gamma_hbm, beta_hbm, out_hbm,
                   idx_all, rows, outb, pos_v, g_v, b_v,
                   sem_g0, sem_g1, sem_o0, sem_o1):
        wid = lax.axis_index("s") * NUM_CORES + lax.axis_index("c")
        row_base = wid * rows_per_worker

        pltpu.sync_copy(seq_hbm.at[pl.ds(row_base, rows_per_worker)], idx_all)
        pltpu.sync_copy(pos_hbm, pos_v)
        pltpu.sync_copy(gamma_hbm, g_v)
        pltpu.sync_copy(beta_hbm, b_v)

        g4 = [g_v[pl.ds(j * LANES, LANES)] for j in range(4)]
        b4 = [b_v[pl.ds(j * LANES, LANES)] for j in range(4)]

        sems_g = (sem_g0, sem_g1)
        sems_o = (sem_o0, sem_o1)

        def gather_copies(r, buf, sem):
            return [
                pltpu.make_async_copy(
                    tok_hbm.at[idx_all.at[r, pl.ds(o, n)]],
                    rows.at[buf, pl.ds(o, n)],
                    sem,
                )
                for o, n in segs
            ]

        def fire_gather(r, buf, sem):
            for cp in gather_copies(r, buf, sem):
                cp.start()

        def wait_gather(r, buf, sem):
            for cp in gather_copies(r, buf, sem):
                cp.wait()

        def store_copy(bi, buf, sem):
            return pltpu.make_async_copy(outb.at[buf], out_hbm.at[bi], sem)

        fire_gather(0, 0, sem_g0)

        def outer(g, _):
            for b in range(2):
                r = g * 2 + b
                nb = 1 - b
                if b == 0:
                    fire_gather(r + 1, nb, sems_g[nb])
                else:
                    @pl.when(g < outer_iters - 1)
                    def _():
                        fire_gather(r + 1, nb, sems_g[nb])
                wait_gather(r, b, sems_g[b])

                bi = row_base + r

                @pl.when(g > 0)
                def _():
                    store_copy(bi, b, sems_o[b]).wait()

                @plsc.parallel_loop(0, seqlen, 1, unroll=4)
                def tok_body(t):
                    x = [rows[b, t, pl.ds(j * LANES, LANES)] * SCALE
                         + pos_v[t, pl.ds(j * LANES, LANES)]
                         for j in range(4)]
                    s = (x[0] + x[1]) + (x[2] + x[3])
                    sq = (x[0] * x[0] + x[1] * x[1]) + (x[2] * x[2] + x[3] * x[3])
                    mean = jnp.sum(s) * (1.0 / DIM)
                    var = jnp.sum(sq) * (1.0 / DIM) - mean * mean
                    inv = _rsqrt(var + EPS)
                    cc = mean * inv
                    for j in range(4):
                        outb[b, t, pl.ds(j * LANES, LANES)] = (
                            (x[j] * inv - cc) * g4[j] + b4[j]
                        )

                store_copy(bi, b, sems_o[b]).start()
            return 0

        lax.fori_loop(0, outer_iters, outer, 0)

        for b in range(2):
            store_copy(row_base + b, b, sems_o[b]).wait()

    return emb_kernel


@jax.jit
def kernel(seq, tok_table, pos_table, gamma, beta):
    b, s = seq.shape
    return _make_kernel(b, s)(
        seq.astype(jnp.int32), tok_table, pos_table, gamma, beta
    )


# speedup vs baseline: 1.3769x; 1.0044x over previous
import functools

import jax
import jax.numpy as jnp
from jax import lax
from jax.experimental import pallas as pl
from jax.experimental.pallas import tpu as pltpu
from jax.experimental.pallas import tpu_sc as plsc

DIM = 64
NUM_CORES = 2
NUM_SUBCORES = 16
NUM_WORKERS = NUM_CORES * NUM_SUBCORES
LANES = 16
IDX_MAX = 128
SCALE = 8.0
EPS = 1e-5


def _rsqrt(v):
    i = lax.bitcast_convert_type(v, jnp.int32)
    i = 0x5F3759DF - lax.shift_right_logical(i, 1)
    y = lax.bitcast_convert_type(i, jnp.float32)
    half = 0.5 * v
    for _ in range(3):
        y = y * (1.5 - half * y * y)
    return y


def _make_kernel(batch, seqlen):
    rows_per_worker = batch // NUM_WORKERS
    outer_iters = rows_per_worker // 2
    segs = []
    off = 0
    while off < seqlen:
        n = min(IDX_MAX, seqlen - off)
        segs.append((off, n))
        off += n
    mesh = plsc.VectorSubcoreMesh(core_axis_name="c", subcore_axis_name="s")

    @functools.partial(
        pl.kernel,
        mesh=mesh,
        compiler_params=pltpu.CompilerParams(
            needs_layout_passes=False, use_tc_tiling_on_sc=False
        ),
        out_type=jax.ShapeDtypeStruct((batch, seqlen, DIM), jnp.float32),
        scratch_types=[
            pltpu.VMEM((rows_per_worker, seqlen), jnp.int32),
            pltpu.VMEM((2, seqlen, DIM), jnp.float32),
            pltpu.VMEM((2, seqlen, DIM), jnp.float32),
            pltpu.VMEM((seqlen, DIM), jnp.float32),
            pltpu.VMEM((DIM,), jnp.float32),
            pltpu.VMEM((DIM,), jnp.float32),
            pltpu.SemaphoreType.DMA,
            pltpu.SemaphoreType.DMA,
            pltpu.SemaphoreType.DMA,
            pltpu.SemaphoreType.DMA,
        ],
    )
    def emb_kernel(seq_hbm, tok_hbm, pos_hbm, gamma_hbm, beta_hbm, out_hbm,
                   idx_all, rows, outb, pos_v, g_v, b_v,
                   sem_g0, sem_g1, sem_o0, sem_o1):
        wid = lax.axis_index("s") * NUM_CORES + lax.axis_index("c")
        row_base = wid * rows_per_worker

        pltpu.sync_copy(seq_hbm.at[pl.ds(row_base, rows_per_worker)], idx_all)
        pltpu.sync_copy(pos_hbm, pos_v)
        pltpu.sync_copy(gamma_hbm, g_v)
        pltpu.sync_copy(beta_hbm, b_v)

        g4 = [g_v[pl.ds(j * LANES, LANES)] for j in range(4)]
        b4 = [b_v[pl.ds(j * LANES, LANES)] for j in range(4)]

        sems_g = (sem_g0, sem_g1)
        sems_o = (sem_o0, sem_o1)

        def gather_copies(r, buf, sem):
            return [
                pltpu.make_async_copy(
                    tok_hbm.at[idx_all.at[r, pl.ds(o, n)]],
                    rows.at[buf, pl.ds(o, n)],
                    sem,
                )
                for o, n in segs
            ]

        def fire_gather(r, buf, sem):
            for cp in gather_copies(r, buf, sem):
                cp.start()

        def wait_gather(r, buf, sem):
            for cp in gather_copies(r, buf, sem):
                cp.wait()

        def store_copy(bi, buf, sem):
            return pltpu.make_async_copy(outb.at[buf], out_hbm.at[bi], sem)

        fire_gather(0, 0, sem_g0)

        def outer(g, _):
            for b in range(2):
                r = g * 2 + b
                nb = 1 - b
                if b == 0:
                    fire_gather(r + 1, nb, sems_g[nb])
                else:
                    @pl.when(g < outer_iters - 1)
                    def _():
                        fire_gather(r + 1, nb, sems_g[nb])
                wait_gather(r, b, sems_g[b])

                bi = row_base + r

                @pl.when(g > 0)
                def _():
                    store_copy(bi, b, sems_o[b]).wait()

                @plsc.parallel_loop(0, seqlen, 1, unroll=2)
                def tok_body(t):
                    x = [rows[b, t, pl.ds(j * LANES, LANES)] * SCALE
                         + pos_v[t, pl.ds(j * LANES, LANES)]
                         for j in range(4)]
                    s = (x[0] + x[1]) + (x[2] + x[3])
                    sq = (x[0] * x[0] + x[1] * x[1]) + (x[2] * x[2] + x[3] * x[3])
                    mean = jnp.sum(s) * (1.0 / DIM)
                    var = jnp.sum(sq) * (1.0 / DIM) - mean * mean
                    inv = _rsqrt(var + EPS)
                    cc = mean * inv
                    for j in range(4):
                        outb[b, t, pl.ds(j * LANES, LANES)] = (
                            (x[j] * inv - cc) * g4[j] + b4[j]
                        )

                store_copy(bi, b, sems_o[b]).start()
            return 0

        lax.fori_loop(0, outer_iters, outer, 0)

        for b in range(2):
            store_copy(row_base + b, b, sems_o[b]).wait()

    return emb_kernel


@jax.jit
def kernel(seq, tok_table, pos_table, gamma, beta):
    b, s = seq.shape
    return _make_kernel(b, s)(
        seq.astype(jnp.int32), tok_table, pos_table, gamma, beta
    )
